# baseline (speedup 1.0000x reference)
import functools

import jax
import jax.numpy as jnp
from jax.experimental import pallas as pl
from jax.experimental.pallas import tpu as pltpu

_LEAKY_SLOPE = 0.2
_BN_EPS = 1e-5
_CB = 8
_NB_NODES = 4
_DEG = _NB_NODES + 1

_CELL_ARCH = (
    (0, 1, 0), (0, 2, 1), (1, 2, 2), (1, 3, 3), (2, 3, 4), (0, 4, 5),
    (3, 4, 6),
)


def _build_links():
    d = {}
    for src, dst, w in _CELL_ARCH:
        d.setdefault(dst, []).append((src, w))
    return tuple(tuple(d[k]) for k in range(1, _NB_NODES + 1))


_LINKS = _build_links()

_ROW_BV, _ROW_GV, _ROW_BETAV, _ROW_BS, _ROW_GE, _ROW_BETAE = range(6)


def _fold_weights(weight, Wv, d):
    zero = jnp.zeros((), jnp.float32)
    coeffs = [[1.0] + [zero] * (_DEG - 1)]
    for dst_links in _LINKS:
        acc = [zero] * _DEG
        for s, w in dst_links:
            w1 = weight[w, 1]
            w2 = weight[w, 2]
            c = coeffs[s]
            for j in range(_DEG):
                acc[j] = acc[j] + w1 * c[j]
                if j + 1 < _DEG:
                    acc[j + 1] = acc[j + 1] + w2 * c[j]
        coeffs.append(acc)
    cmat = jnp.stack([jnp.stack([jnp.asarray(x, jnp.float32) for x in row])
                      for row in coeffs[1:]])
    wv3 = Wv.reshape(_NB_NODES, d, Wv.shape[1])
    wk = jnp.einsum("kj,kxy->jxy", cmat, wv3)
    return wk.reshape(_DEG * d, Wv.shape[1])


def _kernel_body(a_ref, g_ref, v_ref, e_ref, wk_ref, ws_ref, pp_ref,
                 vout_ref, eout_ref, *, node_dim, edge_dim, slope, eps):
    d, de = node_dim, edge_dim
    m = e_ref.shape[1]

    bv = pp_ref[_ROW_BV:_ROW_BV + 1, :d]
    gv = pp_ref[_ROW_GV:_ROW_GV + 1, :d]
    betav = pp_ref[_ROW_BETAV:_ROW_BETAV + 1, :d]
    bs = pp_ref[_ROW_BS:_ROW_BS + 1, :de]
    ge = pp_ref[_ROW_GE:_ROW_GE + 1, :de]
    betae = pp_ref[_ROW_BETAE:_ROW_BETAE + 1, :de]

    wk = [wk_ref[j * d:(j + 1) * d, :] for j in range(_DEG)]
    ws0 = ws_ref[0:d, :]
    ws1 = ws_ref[d:d + de, :]
    ws2 = ws_ref[d + de:2 * d + de, :]

    def bn_leaky_res(h, g, b, res):
        inv_n = 1.0 / h.shape[0]
        s1 = jnp.sum(h, axis=0, keepdims=True)
        s2 = jnp.sum(h * h, axis=0, keepdims=True)
        mean = s1 * inv_n
        var = s2 * inv_n - mean * mean
        alpha = jax.lax.rsqrt(var + eps) * g
        beta = b - mean * alpha
        hn = h * alpha + beta
        return jnp.where(hn >= 0, hn, slope * hn) + res

    kprev = [v_ref[c] for c in range(_CB)]
    v_lin = [jnp.dot(kprev[c], wk[0], preferred_element_type=jnp.float32)
             + bv for c in range(_CB)]
    for j in range(1, _DEG):
        knext = [jnp.dot(a_ref[c], kprev[c],
                         preferred_element_type=jnp.float32)
                 for c in range(_CB)]
        for c in range(_CB):
            v_lin[c] = v_lin[c] + jnp.dot(knext[c], wk[j],
                                          preferred_element_type=jnp.float32)
        kprev = knext

    vg = [jnp.dot(g_ref[c], v_lin[c], preferred_element_type=jnp.float32)
          for c in range(_CB)]

    e_lin = []
    for c in range(_CB):
        e_in = e_ref[c]
        e_act = jnp.where(e_in >= 0, e_in, slope * e_in)
        e_lin.append(
            jnp.dot(vg[c][:m], ws0, preferred_element_type=jnp.float32)
            + jnp.dot(e_act, ws1, preferred_element_type=jnp.float32)
            + jnp.dot(vg[c][m:], ws2, preferred_element_type=jnp.float32)
            + bs)

    for c in range(_CB):
        vout_ref[c] = bn_leaky_res(v_lin[c], gv, betav, v_ref[c])
        eout_ref[c] = bn_leaky_res(e_lin[c], ge, betae, e_ref[c])


def _pack_params(bv, gv, betav, bs, ge, betae, d, de):
    pp = jnp.zeros((8, 128), jnp.float32)
    pp = pp.at[_ROW_BV, :d].set(bv.reshape(-1))
    pp = pp.at[_ROW_GV, :d].set(gv.reshape(-1))
    pp = pp.at[_ROW_BETAV, :d].set(betav.reshape(-1))
    pp = pp.at[_ROW_BS, :de].set(bs.reshape(-1))
    pp = pp.at[_ROW_GE, :de].set(ge.reshape(-1))
    pp = pp.at[_ROW_BETAE, :de].set(betae.reshape(-1))
    return pp


def kernel(Wv, bv, Ws, bs, gv, betav, ge, betae, weight,
           a_mean_b, s_gather_b, v_in, e_in):
    b, n, d = v_in.shape
    _, m, de = e_in.shape
    pp = _pack_params(bv, gv, betav, bs, ge, betae, d, de)
    wk_stack = _fold_weights(weight, Wv, d)

    body = functools.partial(_kernel_body, node_dim=d, edge_dim=de,
                             slope=_LEAKY_SLOPE, eps=_BN_EPS)

    in_specs = [
        pl.BlockSpec((_CB, n, n), lambda i: (i, 0, 0)),
        pl.BlockSpec((_CB, 2 * m, n), lambda i: (i, 0, 0)),
        pl.BlockSpec((_CB, n, d), lambda i: (i, 0, 0)),
        pl.BlockSpec((_CB, m, de), lambda i: (i, 0, 0)),
        pl.BlockSpec(wk_stack.shape, lambda i: (0, 0)),
        pl.BlockSpec(Ws.shape, lambda i: (0, 0)),
        pl.BlockSpec((8, 128), lambda i: (0, 0)),
    ]
    out_specs = (
        pl.BlockSpec((_CB, n, d), lambda i: (i, 0, 0)),
        pl.BlockSpec((_CB, m, de), lambda i: (i, 0, 0)),
    )
    out_shape = (jax.ShapeDtypeStruct((b, n, d), jnp.float32),
                 jax.ShapeDtypeStruct((b, m, de), jnp.float32))

    flops_per_cell = (2 * 4 * n * n * d + 2 * n * (_DEG * d) * d
                      + 2 * (2 * m) * n * d + 2 * m * (2 * d + de) * de
                      + 12 * (n * d + m * de))
    bytes_accessed = 4 * (a_mean_b.size + s_gather_b.size + v_in.size
                          + e_in.size + Wv.size + Ws.size + 8 * 128
                          + b * n * d + b * m * de)

    return pl.pallas_call(
        body,
        grid=(b // _CB,),
        in_specs=in_specs,
        out_specs=out_specs,
        out_shape=out_shape,
        compiler_params=pltpu.CompilerParams(
            dimension_semantics=("parallel",)),
        cost_estimate=pl.CostEstimate(
            flops=int(b * flops_per_cell),
            transcendentals=int(b * (d + de)),
            bytes_accessed=int(bytes_accessed)),
    )(a_mean_b, s_gather_b, v_in, e_in, wk_stack, Ws, pp)
